# Initial kernel scaffold; baseline (speedup 1.0000x reference)
import jax
import jax.numpy as jnp
from jax.experimental import pallas as pl


def kernel(inputs):
    raise NotImplementedError("write your pallas kernel here")


# speedup vs baseline: 1.0368x; 1.0368x over previous
import jax
import jax.numpy as jnp
from jax.experimental import pallas as pl


def _slice_body(in_ref, out_ref):
    out_ref[...] = in_ref[:, 7, :]


def kernel(inputs):
    B, S, D = inputs.shape
    return pl.pallas_call(
        _slice_body,
        grid=(1,),
        in_specs=[
            pl.BlockSpec((B, 8, D), lambda i: (0, S // 8 - 1, 0)),
        ],
        out_specs=pl.BlockSpec((B, D), lambda i: (0, 0)),
        out_shape=jax.ShapeDtypeStruct((B, D), inputs.dtype),
    )(inputs)
